# Initial kernel scaffold; baseline (speedup 1.0000x reference)
import jax
import jax.numpy as jnp
from jax.experimental import pallas as pl


def kernel(tokens, table):
    raise NotImplementedError("write your pallas kernel here")


# speedup vs baseline: 5.4854x; 5.4854x over previous
import functools
import math

import jax
import jax.numpy as jnp
from jax import lax
from jax.experimental import pallas as pl
from jax.experimental.pallas import tpu as pltpu
from jax.experimental.pallas import tpu_sc as plsc

VOCAB_ROWS = 100000
EMB_DIM = 128
SCALE = math.sqrt(float(EMB_DIM))

NUM_CORES = 2
NUM_SUBCORES = 16
NW = NUM_CORES * NUM_SUBCORES

CHUNK = 128


def _scale_body(t_ref, o_ref):
    o_ref[...] = t_ref[...] * SCALE


def _scale_table(table):
    rows = table.shape[0]
    block = 1000
    return pl.pallas_call(
        _scale_body,
        grid=(rows // block,),
        in_specs=[pl.BlockSpec((block, EMB_DIM), lambda i: (i, 0))],
        out_specs=pl.BlockSpec((block, EMB_DIM), lambda i: (i, 0)),
        out_shape=jax.ShapeDtypeStruct((rows, EMB_DIM), jnp.float32),
    )(table)


def _make_gather(n_tokens):
    assert n_tokens % (NW * CHUNK) == 0
    bpw = n_tokens // NW
    n_chunks = bpw // CHUNK

    mesh = plsc.VectorSubcoreMesh(core_axis_name="c", subcore_axis_name="s")

    @functools.partial(
        pl.kernel,
        mesh=mesh,
        out_type=jax.ShapeDtypeStruct((n_tokens, EMB_DIM), jnp.float32),
        scratch_types=[
            pltpu.VMEM((n_chunks, CHUNK), jnp.int32),
            pltpu.VMEM((CHUNK, EMB_DIM), jnp.float32),
            pltpu.SemaphoreType.DMA,
        ],
    )
    def gather_kernel(idx_hbm, table_hbm, out_hbm, idx_v, buf, sem):
        wid = lax.axis_index("s") * NUM_CORES + lax.axis_index("c")
        base = wid * bpw
        pltpu.sync_copy(idx_hbm.at[wid], idx_v)

        def body(g, carry):
            pltpu.async_copy(table_hbm.at[idx_v.at[g]], buf, sem).wait()
            pltpu.sync_copy(buf, out_hbm.at[pl.ds(base + g * CHUNK, CHUNK)])
            return carry

        lax.fori_loop(0, n_chunks, body, 0)

    return gather_kernel


def kernel(tokens, table):
    n_tokens = tokens.shape[0] * tokens.shape[1]
    idx = tokens.reshape(NW, n_tokens // (NW * CHUNK), CHUNK).astype(jnp.int32)
    scaled = _scale_table(table)
    out = _make_gather(n_tokens)(idx, scaled)
    return out.reshape(tokens.shape[0], tokens.shape[1], EMB_DIM)


# speedup vs baseline: 7.5564x; 1.3776x over previous
import functools
import math

import jax
import jax.numpy as jnp
from jax import lax
from jax.experimental import pallas as pl
from jax.experimental.pallas import tpu as pltpu
from jax.experimental.pallas import tpu_sc as plsc

VOCAB_ROWS = 100000
EMB_DIM = 128
SCALE = math.sqrt(float(EMB_DIM))

NUM_CORES = 2
NUM_SUBCORES = 16
NW = NUM_CORES * NUM_SUBCORES

CHUNK = 128


def _scale_body(t_ref, o_ref):
    o_ref[...] = t_ref[...] * SCALE


def _scale_table(table):
    rows = table.shape[0]
    block = 1000
    return pl.pallas_call(
        _scale_body,
        grid=(rows // block,),
        in_specs=[pl.BlockSpec((block, EMB_DIM), lambda i: (i, 0))],
        out_specs=pl.BlockSpec((block, EMB_DIM), lambda i: (i, 0)),
        out_shape=jax.ShapeDtypeStruct((rows, EMB_DIM), jnp.float32),
    )(table)


def _make_gather(n_tokens):
    assert n_tokens % (NW * CHUNK) == 0
    bpw = n_tokens // NW
    n_chunks = bpw // CHUNK

    mesh = plsc.VectorSubcoreMesh(core_axis_name="c", subcore_axis_name="s")

    n_pairs = n_chunks // 2
    assert n_chunks % 2 == 0

    @functools.partial(
        pl.kernel,
        mesh=mesh,
        out_type=jax.ShapeDtypeStruct((n_tokens, EMB_DIM), jnp.float32),
        scratch_types=[
            pltpu.VMEM((n_chunks, CHUNK), jnp.int32),
            pltpu.VMEM((CHUNK, EMB_DIM), jnp.float32),
            pltpu.VMEM((CHUNK, EMB_DIM), jnp.float32),
            pltpu.SemaphoreType.DMA,
            pltpu.SemaphoreType.DMA,
        ],
    )
    def gather_kernel(idx_hbm, table_hbm, out_hbm, idx_v, buf0, buf1, sg0, sg1):
        wid = lax.axis_index("s") * NUM_CORES + lax.axis_index("c")
        base = wid * bpw
        pltpu.sync_copy(idx_hbm.at[wid], idx_v)

        pltpu.async_copy(table_hbm.at[idx_v.at[0]], buf0, sg0)

        def body(h, carry):
            g0 = 2 * h
            g1 = g0 + 1
            pltpu.async_copy(table_hbm.at[idx_v.at[g1]], buf1, sg1)
            pltpu.make_async_copy(table_hbm.at[idx_v.at[g0]], buf0, sg0).wait()
            pltpu.sync_copy(buf0, out_hbm.at[pl.ds(base + g0 * CHUNK, CHUNK)])

            @pl.when(h + 1 < n_pairs)
            def _():
                pltpu.async_copy(table_hbm.at[idx_v.at[g0 + 2]], buf0, sg0)

            pltpu.make_async_copy(table_hbm.at[idx_v.at[g1]], buf1, sg1).wait()
            pltpu.sync_copy(buf1, out_hbm.at[pl.ds(base + g1 * CHUNK, CHUNK)])
            return carry

        lax.fori_loop(0, n_pairs, body, 0)

    return gather_kernel


def kernel(tokens, table):
    n_tokens = tokens.shape[0] * tokens.shape[1]
    idx = tokens.reshape(NW, n_tokens // (NW * CHUNK), CHUNK).astype(jnp.int32)
    scaled = _scale_table(table)
    out = _make_gather(n_tokens)(idx, scaled)
    return out.reshape(tokens.shape[0], tokens.shape[1], EMB_DIM)
